# baseline (speedup 1.0000x reference)
import functools

import jax
import jax.numpy as jnp
from jax import lax
from jax.experimental import pallas as pl
from jax.experimental.pallas import tpu as pltpu
from jax.experimental.pallas import tpu_sc as plsc

NC = 2
NS = 16
NW = NC * NS
LANES = 16
CB = 128

_mesh = plsc.VectorSubcoreMesh(
    core_axis_name="c", subcore_axis_name="s", num_cores=NC, num_subcores=NS)


def _deg_body(n_pad, nchunk, dst_hbm, out_hbm, dst_v, hist_v):
    c = lax.axis_index("c")
    s = lax.axis_index("s")
    wid = s * NC + c

    def zero(i, _):
        hist_v[pl.ds(i * LANES, LANES)] = jnp.zeros((LANES,), jnp.float32)
        return 0
    lax.fori_loop(0, n_pad // LANES, zero, 0)

    pltpu.sync_copy(dst_hbm.at[pl.ds(wid * nchunk, nchunk)], dst_v)

    unroll = 4
    vregs_per_row = CB // LANES

    def acc(i, _):
        j = i // (vregs_per_row // unroll)
        k = (i % (vregs_per_row // unroll)) * (unroll * LANES)
        for u in range(unroll):
            idx = dst_v[j, pl.ds(k + u * LANES, LANES)]
            cnt, last = plsc.scan_count(idx)
            plsc.addupdate_scatter(
                hist_v, [idx], cnt.astype(jnp.float32), mask=last)
        return 0
    lax.fori_loop(0, nchunk * vregs_per_row // unroll, acc, 0)

    pltpu.sync_copy(hist_v, out_hbm.at[wid])


def _agg_body(n_pad, n0, n1, feat, dummy,
              u_hbm, src_hbm, dst_hbm, out_hbm,
              src_v, dst_v, rows_v, zbuf, vshared, gsem):
    c = lax.axis_index("c")
    s = lax.axis_index("s")
    npt = n_pad // NS
    nmax = max(n0, n1)
    start = jnp.where(c == 0, s * n0, NS * n0 + s * n1)
    cnt = jnp.where(c == 0, n0, n1)

    def zero(i, _):
        for kk in range(feat // LANES):
            zbuf[i, pl.ds(kk * LANES, LANES)] = jnp.zeros((LANES,), jnp.float32)
        return 0
    lax.fori_loop(0, npt, zero, 0)
    pltpu.sync_copy(zbuf, vshared.at[pl.ds(s * npt, npt)])

    pltpu.sync_copy(src_hbm.at[pl.ds(start, nmax + 1)], src_v)
    pltpu.sync_copy(dst_hbm.at[pl.ds(start, nmax + 1)], dst_v)

    plsc.subcore_barrier()

    pltpu.async_copy(u_hbm.at[src_v.at[0]], rows_v.at[0], gsem)

    def step(t2, _):
        for bi in range(2):
            t = t2 * 2 + bi
            pltpu.make_async_copy(
                u_hbm.at[src_v.at[t]], rows_v.at[bi], gsem).wait()
            pltpu.async_copy(
                u_hbm.at[src_v.at[t + 1]], rows_v.at[1 - bi], gsem)
            pltpu.sync_copy(rows_v.at[bi], vshared.at[dst_v.at[t]], add=True)
        return 0
    lax.fori_loop(0, cnt // 2, step, 0)
    pltpu.make_async_copy(u_hbm.at[src_v.at[cnt]], rows_v.at[0], gsem).wait()

    plsc.subcore_barrier()
    pltpu.sync_copy(vshared.at[pl.ds(s * npt, npt)],
                    out_hbm.at[c, pl.ds(s * npt, npt)])


def _make_deg(n_pad, nchunk):
    return pl.kernel(
        functools.partial(_deg_body, n_pad, nchunk),
        out_type=jax.ShapeDtypeStruct((NW, n_pad), jnp.float32),
        mesh=_mesh,
        compiler_params=pltpu.CompilerParams(needs_layout_passes=False),
        scratch_types=[
            pltpu.VMEM((nchunk, CB), jnp.int32),
            pltpu.VMEM((n_pad,), jnp.float32),
        ],
    )


def _make_agg(n_pad, n0, n1, feat, dummy):
    nmax = max(n0, n1)
    return pl.kernel(
        functools.partial(_agg_body, n_pad, n0, n1, feat, dummy),
        out_type=jax.ShapeDtypeStruct((NC, n_pad, feat), jnp.float32),
        mesh=_mesh,
        compiler_params=pltpu.CompilerParams(use_tc_tiling_on_sc=False),
        scratch_types=[
            pltpu.VMEM((nmax + 1, CB), jnp.int32),
            pltpu.VMEM((nmax + 1, CB), jnp.int32),
            pltpu.VMEM((2, CB, feat), jnp.float32),
            pltpu.VMEM((n_pad // NS, feat), jnp.float32),
            pltpu.VMEM_SHARED((n_pad, feat), jnp.float32),
            pltpu.SemaphoreType.DMA,
        ],
    )


def _dense1_body(x_ref, w1_ref, hist_ref, u1_ref, dinv_ref):
    deg = jnp.sum(hist_ref[...], axis=0) + 1.0
    dinv = lax.rsqrt(deg)[:, None]
    h = jnp.dot(x_ref[...], w1_ref[...], preferred_element_type=jnp.float32)
    u1_ref[...] = h * dinv
    dinv_ref[...] = dinv


def _dense2_body(v1_ref, u1_ref, dinv_ref, b1_ref, w_ref):
    dinv = dinv_ref[...]
    agg = (v1_ref[0] + v1_ref[1] + u1_ref[...]) * dinv + b1_ref[...]
    h1 = jnp.maximum(agg, 0.0)
    w_ref[...] = h1 * dinv


def _final_body(n_nodes, v2_ref, w_ref, dinv_ref, b2_ref, w2_ref, wfc_ref,
                bfc_ref, out_ref):
    agg1 = (v2_ref[0] + v2_ref[1] + w_ref[...]) * dinv_ref[...]
    agg = jnp.dot(
        agg1, w2_ref[...], preferred_element_type=jnp.float32) + b2_ref[...]
    h2 = jnp.maximum(agg, 0.0)
    row = lax.broadcasted_iota(jnp.int32, h2.shape, 0)
    h2 = jnp.where(row < n_nodes, h2, 0.0)
    pooled = jnp.sum(h2, axis=0, keepdims=True)
    z = jnp.dot(pooled, wfc_ref[...], preferred_element_type=jnp.float32)
    z = z + bfc_ref[...] * jnp.float32(n_nodes)
    m = jnp.max(z, axis=1, keepdims=True)
    lse = jnp.log(jnp.sum(jnp.exp(z - m), axis=1, keepdims=True)) + m
    out_ref[...] = z - lse


def _tc(body, out_shape):
    return pl.pallas_call(body, out_shape=out_shape)


def kernel(x, edge_index, W1, b1, W2, b2, W_fc, b_fc):
    n_nodes, d_feat = x.shape
    f1 = W1.shape[1]
    f2 = W2.shape[1]
    n_edges = edge_index.shape[1]

    n_pad = ((n_nodes + 1 + NS * 8 - 1) // (NS * 8)) * (NS * 8)
    nchunk = -(-n_edges // (NW * CB))
    nchunk += nchunk % 2
    delta = 18
    n0, n1 = nchunk + delta, nchunk - delta
    nmax = max(n0, n1)
    tt = NS * (n0 + n1) + nmax + 1
    dummy = n_nodes

    ei = edge_index.astype(jnp.int32)
    pad = jnp.full((tt * CB - n_edges,), dummy, jnp.int32)
    src = jnp.concatenate([ei[0], pad]).reshape(tt, CB)
    dst = jnp.concatenate([ei[1], pad]).reshape(tt, CB)
    x_pad = jnp.pad(x, ((0, n_pad - n_nodes), (0, 0)))
    hist = _make_deg(n_pad, (NS * (n0 + n1)) // NW)(dst)

    u1, dinv = _tc(
        _dense1_body,
        (jax.ShapeDtypeStruct((n_pad, f1), jnp.float32),
         jax.ShapeDtypeStruct((n_pad, 1), jnp.float32)),
    )(x_pad, W1, hist)

    agg = _make_agg(n_pad, n0, n1, f1, dummy)
    v1 = agg(u1, src, dst)

    w = _tc(
        _dense2_body,
        jax.ShapeDtypeStruct((n_pad, f1), jnp.float32),
    )(v1, u1, dinv, b1.reshape(1, f1))

    v2 = agg(w, src, dst)

    out = _tc(
        functools.partial(_final_body, n_nodes),
        jax.ShapeDtypeStruct((1, W_fc.shape[1]), jnp.float32),
    )(v2, w, dinv, b2.reshape(1, f2), W2, W_fc,
      b_fc.reshape(1, W_fc.shape[1]))
    return out


# speedup vs baseline: 1.0015x; 1.0015x over previous
import functools

import jax
import jax.numpy as jnp
from jax import lax
from jax.experimental import pallas as pl
from jax.experimental.pallas import tpu as pltpu
from jax.experimental.pallas import tpu_sc as plsc

NC = 2
NS = 16
NW = NC * NS
LANES = 16
CB = 128

_mesh = plsc.VectorSubcoreMesh(
    core_axis_name="c", subcore_axis_name="s", num_cores=NC, num_subcores=NS)


def _deg_body(n_pad, nchunk, dst_hbm, out_hbm, dst_v, hist_v):
    c = lax.axis_index("c")
    s = lax.axis_index("s")
    wid = s * NC + c

    def zero(i, _):
        hist_v[pl.ds(i * LANES, LANES)] = jnp.zeros((LANES,), jnp.float32)
        return 0
    lax.fori_loop(0, n_pad // LANES, zero, 0)

    pltpu.sync_copy(dst_hbm.at[pl.ds(wid * nchunk, nchunk)], dst_v)

    unroll = 4
    vregs_per_row = CB // LANES

    def acc(i, _):
        j = i // (vregs_per_row // unroll)
        k = (i % (vregs_per_row // unroll)) * (unroll * LANES)
        for u in range(unroll):
            idx = dst_v[j, pl.ds(k + u * LANES, LANES)]
            cnt, last = plsc.scan_count(idx)
            plsc.addupdate_scatter(
                hist_v, [idx], cnt.astype(jnp.float32), mask=last)
        return 0
    lax.fori_loop(0, nchunk * vregs_per_row // unroll, acc, 0)

    pltpu.sync_copy(hist_v, out_hbm.at[wid])


def _agg_body(n_pad, n0, n1, feat, dummy,
              u_hbm, src_hbm, dst_hbm, out_hbm,
              src_v, dst_v, rows_v, zbuf, vshared, gsem):
    c = lax.axis_index("c")
    s = lax.axis_index("s")
    npt = n_pad // NS
    nmax = max(n0, n1)
    start = jnp.where(c == 0, s * n0, NS * n0 + s * n1)
    cnt = jnp.where(c == 0, n0, n1)

    def zero(i, _):
        for kk in range(feat // LANES):
            zbuf[i, pl.ds(kk * LANES, LANES)] = jnp.zeros((LANES,), jnp.float32)
        return 0
    lax.fori_loop(0, npt, zero, 0)
    pltpu.sync_copy(zbuf, vshared.at[pl.ds(s * npt, npt)])

    pltpu.sync_copy(src_hbm.at[pl.ds(start, nmax + 1)], src_v)
    pltpu.sync_copy(dst_hbm.at[pl.ds(start, nmax + 1)], dst_v)

    plsc.subcore_barrier()

    pltpu.async_copy(u_hbm.at[src_v.at[0]], rows_v.at[0], gsem)

    def step(t2, _):
        for bi in range(2):
            t = t2 * 2 + bi
            pltpu.make_async_copy(
                u_hbm.at[src_v.at[t]], rows_v.at[bi], gsem).wait()
            pltpu.async_copy(
                u_hbm.at[src_v.at[t + 1]], rows_v.at[1 - bi], gsem)
            pltpu.sync_copy(rows_v.at[bi], vshared.at[dst_v.at[t]], add=True)
        return 0
    lax.fori_loop(0, cnt // 2, step, 0)
    pltpu.make_async_copy(u_hbm.at[src_v.at[cnt]], rows_v.at[0], gsem).wait()

    plsc.subcore_barrier()
    pltpu.sync_copy(vshared.at[pl.ds(s * npt, npt)],
                    out_hbm.at[c, pl.ds(s * npt, npt)])


def _make_deg(n_pad, nchunk):
    return pl.kernel(
        functools.partial(_deg_body, n_pad, nchunk),
        out_type=jax.ShapeDtypeStruct((NW, n_pad), jnp.float32),
        mesh=_mesh,
        compiler_params=pltpu.CompilerParams(needs_layout_passes=False),
        scratch_types=[
            pltpu.VMEM((nchunk, CB), jnp.int32),
            pltpu.VMEM((n_pad,), jnp.float32),
        ],
    )


def _make_agg(n_pad, n0, n1, feat, dummy):
    nmax = max(n0, n1)
    return pl.kernel(
        functools.partial(_agg_body, n_pad, n0, n1, feat, dummy),
        out_type=jax.ShapeDtypeStruct((NC, n_pad, feat), jnp.float32),
        mesh=_mesh,
        compiler_params=pltpu.CompilerParams(use_tc_tiling_on_sc=False),
        scratch_types=[
            pltpu.VMEM((nmax + 1, CB), jnp.int32),
            pltpu.VMEM((nmax + 1, CB), jnp.int32),
            pltpu.VMEM((2, CB, feat), jnp.float32),
            pltpu.VMEM((n_pad // NS, feat), jnp.float32),
            pltpu.VMEM_SHARED((n_pad, feat), jnp.float32),
            pltpu.SemaphoreType.DMA,
        ],
    )


def _dense1_body(x_ref, w1_ref, hist_ref, u1_ref, dinv_ref):
    deg = jnp.sum(hist_ref[...], axis=0) + 1.0
    dinv = (deg ** -0.5)[:, None]
    h = jnp.dot(x_ref[...], w1_ref[...], preferred_element_type=jnp.float32)
    u1_ref[...] = h * dinv
    dinv_ref[...] = dinv


def _dense2_body(v1_ref, u1_ref, dinv_ref, b1_ref, w_ref):
    dinv = dinv_ref[...]
    agg = (v1_ref[0] + v1_ref[1] + u1_ref[...]) * dinv + b1_ref[...]
    h1 = jnp.maximum(agg, 0.0)
    w_ref[...] = h1 * dinv


def _final_body(n_nodes, v2_ref, w_ref, dinv_ref, b2_ref, w2_ref, wfc_ref,
                bfc_ref, out_ref):
    agg1 = (v2_ref[0] + v2_ref[1] + w_ref[...]) * dinv_ref[...]
    agg = jnp.dot(
        agg1, w2_ref[...], preferred_element_type=jnp.float32) + b2_ref[...]
    h2 = jnp.maximum(agg, 0.0)
    row = lax.broadcasted_iota(jnp.int32, h2.shape, 0)
    h2 = jnp.where(row < n_nodes, h2, 0.0)
    pooled = jnp.sum(h2, axis=0, keepdims=True)
    z = jnp.dot(pooled, wfc_ref[...], preferred_element_type=jnp.float32)
    z = z + bfc_ref[...] * jnp.float32(n_nodes)
    m = jnp.max(z, axis=1, keepdims=True)
    lse = jnp.log(jnp.sum(jnp.exp(z - m), axis=1, keepdims=True)) + m
    out_ref[...] = z - lse


def _tc(body, out_shape):
    return pl.pallas_call(body, out_shape=out_shape)


def kernel(x, edge_index, W1, b1, W2, b2, W_fc, b_fc):
    n_nodes, d_feat = x.shape
    f1 = W1.shape[1]
    f2 = W2.shape[1]
    n_edges = edge_index.shape[1]

    n_pad = ((n_nodes + 1 + NS * 8 - 1) // (NS * 8)) * (NS * 8)
    nchunk = -(-n_edges // (NW * CB))
    nchunk += nchunk % 2
    delta = 18
    n0, n1 = nchunk + delta, nchunk - delta
    nmax = max(n0, n1)
    tt = NS * (n0 + n1) + nmax + 1
    dummy = n_nodes

    ei = edge_index.astype(jnp.int32)
    pad = jnp.full((tt * CB - n_edges,), dummy, jnp.int32)
    src = jnp.concatenate([ei[0], pad]).reshape(tt, CB)
    dst = jnp.concatenate([ei[1], pad]).reshape(tt, CB)
    x_pad = jnp.pad(x, ((0, n_pad - n_nodes), (0, 0)))
    hist = _make_deg(n_pad, (NS * (n0 + n1)) // NW)(dst)

    u1, dinv = _tc(
        _dense1_body,
        (jax.ShapeDtypeStruct((n_pad, f1), jnp.float32),
         jax.ShapeDtypeStruct((n_pad, 1), jnp.float32)),
    )(x_pad, W1, hist)

    agg = _make_agg(n_pad, n0, n1, f1, dummy)
    v1 = agg(u1, src, dst)

    w = _tc(
        _dense2_body,
        jax.ShapeDtypeStruct((n_pad, f1), jnp.float32),
    )(v1, u1, dinv, b1.reshape(1, f1))

    v2 = agg(w, src, dst)

    out = _tc(
        functools.partial(_final_body, n_nodes),
        jax.ShapeDtypeStruct((1, W_fc.shape[1]), jnp.float32),
    )(v2, w, dinv, b2.reshape(1, f2), W2, W_fc,
      b_fc.reshape(1, W_fc.shape[1]))
    return out
